# Initial kernel scaffold; baseline (speedup 1.0000x reference)
import jax
import jax.numpy as jnp
from jax.experimental import pallas as pl


def kernel(features, edge_index, weight, edge_weight, W1, b1, W3, b3):
    raise NotImplementedError("write your pallas kernel here")


# speedup vs baseline: 6.4195x; 6.4195x over previous
import functools

import jax
import jax.numpy as jnp
from jax import lax
from jax.experimental import pallas as pl
from jax.experimental.pallas import tpu as pltpu
from jax.experimental.pallas import tpu_sc as plsc

N = 10000
NPAD = 10240
E = 320000
D = 128
HID = 256
NCLS = 64

NCORES = 2
NSUB = 16
NW = NCORES * NSUB
EPW = E // NW
K = 80
NCHUNK = EPW // K
RPT = NPAD // NW
F32 = jnp.float32

_mesh = plsc.VectorSubcoreMesh(
    core_axis_name="c", subcore_axis_name="s",
    num_cores=NCORES, num_subcores=NSUB)


def _worker_id():
  cid = lax.axis_index("c")
  sid = lax.axis_index("s")
  return cid, sid, cid * NSUB + sid


@functools.partial(
    pl.kernel,
    out_type=(jax.ShapeDtypeStruct((NCORES, NPAD), F32),
              jax.ShapeDtypeStruct((NCORES, NPAD), F32)),
    mesh=_mesh,
    scratch_types=[
        pltpu.VMEM((K,), jnp.int32),
        pltpu.VMEM((K,), F32),
        pltpu.VMEM((NPAD // NSUB,), F32),
        pltpu.VMEM_SHARED((NPAD,), F32),
        pltpu.VMEM_SHARED((NPAD,), F32),
    ],
)
def _sc_degrees(src_hbm, dst_hbm, dout_hbm, din_hbm,
                idx_v, ones_v, zline_v, acc_out, acc_in):
  cid, sid, wid = _worker_id()
  zs = jnp.zeros((16,), F32)
  os = jnp.ones((16,), F32)

  def fill(i, _):
    zline_v[pl.ds(i * 16, 16)] = zs
    return 0
  lax.fori_loop(0, (NPAD // NSUB) // 16, fill, 0)
  for i in range(K // 16):
    ones_v[pl.ds(i * 16, 16)] = os

  seg = pl.ds(sid * (NPAD // NSUB), NPAD // NSUB)
  pltpu.sync_copy(zline_v, acc_out.at[seg])
  pltpu.sync_copy(zline_v, acc_in.at[seg])
  plsc.subcore_barrier()

  def chunk(c, _):
    off = wid * EPW + c * K
    pltpu.sync_copy(src_hbm.at[pl.ds(off, K)], idx_v)
    pltpu.sync_copy(ones_v, acc_out.at[idx_v], add=True)
    pltpu.sync_copy(dst_hbm.at[pl.ds(off, K)], idx_v)
    pltpu.sync_copy(ones_v, acc_in.at[idx_v], add=True)
    return 0
  lax.fori_loop(0, NCHUNK, chunk, 0)

  plsc.subcore_barrier()
  pltpu.sync_copy(acc_out.at[seg], dout_hbm.at[cid, seg])
  pltpu.sync_copy(acc_in.at[seg], din_hbm.at[cid, seg])


@functools.partial(
    pl.kernel,
    out_type=jax.ShapeDtypeStruct((NCORES, NPAD, D), F32),
    mesh=_mesh,
    scratch_types=[
        pltpu.VMEM((K,), jnp.int32),
        pltpu.VMEM((K,), jnp.int32),
        pltpu.VMEM((K,), F32),
        pltpu.VMEM((K, D), F32),
        pltpu.VMEM((K, D), F32),
        pltpu.VMEM_SHARED((NPAD, D), F32),
        pltpu.SemaphoreType.DMA,
    ],
)
def _sc_spmm(x_hbm, src_hbm, dst_hbm, ew_hbm, out_hbm,
             sidx_v, didx_v, wv, rows_v, zrows_v, acc, sem):
  cid, sid, wid = _worker_id()
  zs = jnp.zeros((16,), F32)

  def zrow(r, _):
    for c in range(D // 16):
      zrows_v[r, pl.ds(c * 16, 16)] = zs
    return 0
  lax.fori_loop(0, K, zrow, 0)
  for j in range(RPT // K):
    pltpu.sync_copy(zrows_v, acc.at[pl.ds(sid * RPT + j * K, K)])
  plsc.subcore_barrier()

  def chunk(c, _):
    off = wid * EPW + c * K
    pltpu.sync_copy(src_hbm.at[pl.ds(off, K)], sidx_v)
    pltpu.sync_copy(dst_hbm.at[pl.ds(off, K)], didx_v)
    pltpu.sync_copy(ew_hbm.at[pl.ds(off, K)], wv)
    pltpu.async_copy(x_hbm.at[sidx_v], rows_v, sem).wait()

    def scale(g, _):
      wvec = wv[pl.ds(g * 16, 16)]
      for j in range(16):
        r = g * 16 + j
        w = wvec[j]
        for cc in range(D // 16):
          sl = pl.ds(cc * 16, 16)
          rows_v[r, sl] = rows_v[r, sl] * w
      return 0
    lax.fori_loop(0, K // 16, scale, 0)

    pltpu.sync_copy(rows_v, acc.at[didx_v], add=True)
    return 0
  lax.fori_loop(0, NCHUNK, chunk, 0)

  plsc.subcore_barrier()
  seg = pl.ds(sid * RPT, RPT)
  pltpu.sync_copy(acc.at[seg], out_hbm.at[cid, seg])


def _tc_prescale(dout_p, din_p, feat):
  def body(do_ref, di_ref, f_ref, xs_ref, dinv_ref):
    dout = jnp.maximum(do_ref[0, :] + do_ref[1, :], 1.0)
    din = jnp.maximum(di_ref[0, :] + di_ref[1, :], 1.0)
    dinv_o = lax.rsqrt(dout)
    dinv_ref[0, :] = dinv_o
    dinv_ref[1, :] = lax.rsqrt(din)
    xs_ref[...] = f_ref[...] * dinv_o[:, None]

  return pl.pallas_call(
      body,
      out_shape=(jax.ShapeDtypeStruct((NPAD, D), F32),
                 jax.ShapeDtypeStruct((2, NPAD), F32)),
  )(dout_p, din_p, feat)


def _tc_mid(ppart, dinvs, W1, b1, W2):
  blk = NPAD // 8

  def body(pp_ref, dv_ref, w1_ref, b1_ref, w2_ref, out_ref):
    p = pp_ref[0] + pp_ref[1]
    x1 = jnp.dot(p * dv_ref[1, :][:, None], w1_ref[...],
                 preferred_element_type=F32) + b1_ref[...][None, :]
    h = jnp.dot(x1 * dv_ref[0, :][:, None], w2_ref[...],
                preferred_element_type=F32)
    out_ref[...] = h

  return pl.pallas_call(
      body,
      grid=(NPAD // blk,),
      in_specs=[
          pl.BlockSpec((NCORES, blk, D), lambda i: (0, i, 0)),
          pl.BlockSpec((2, blk), lambda i: (0, i)),
          pl.BlockSpec((D, HID), lambda i: (0, 0)),
          pl.BlockSpec((HID,), lambda i: (0,)),
          pl.BlockSpec((HID, D), lambda i: (0, 0)),
      ],
      out_specs=pl.BlockSpec((blk, D), lambda i: (i, 0)),
      out_shape=jax.ShapeDtypeStruct((NPAD, D), F32),
  )(ppart, dinvs, W1, b1, W2)


def _tc_head(qpart, dinvs, W3, b3):
  blk = NPAD // 8

  def body(qp_ref, dv_ref, w3_ref, b3_ref, out_ref):
    q = (qp_ref[0] + qp_ref[1]) * dv_ref[1, :][:, None]
    x = jnp.maximum(q, 0.0)
    z = jnp.dot(x, w3_ref[...], preferred_element_type=F32) + b3_ref[...][None, :]
    z = z - jnp.max(z, axis=1, keepdims=True)
    e = jnp.exp(z)
    out_ref[...] = e / jnp.sum(e, axis=1, keepdims=True)

  return pl.pallas_call(
      body,
      grid=(NPAD // blk,),
      in_specs=[
          pl.BlockSpec((NCORES, blk, D), lambda i: (0, i, 0)),
          pl.BlockSpec((2, blk), lambda i: (0, i)),
          pl.BlockSpec((D, NCLS), lambda i: (0, 0)),
          pl.BlockSpec((NCLS,), lambda i: (0,)),
      ],
      out_specs=pl.BlockSpec((blk, NCLS), lambda i: (i, 0)),
      out_shape=jax.ShapeDtypeStruct((NPAD, NCLS), F32),
  )(qpart, dinvs, W3, b3)


@jax.jit
def kernel(features, edge_index, weight, edge_weight, W1, b1, W3, b3):
  src = edge_index[0]
  dst = edge_index[1]
  feat = jnp.zeros((NPAD, D), F32).at[:N, :].set(features)

  dout_p, din_p = _sc_degrees(src, dst)
  xs, dinvs = _tc_prescale(dout_p, din_p, feat)
  ppart = _sc_spmm(xs, src, dst, edge_weight)
  h = _tc_mid(ppart, dinvs, W1, b1, weight)
  qpart = _sc_spmm(h, src, dst, edge_weight)
  out = _tc_head(qpart, dinvs, W3, b3)
  return out[:N, :]
